# baseline (speedup 1.0000x reference)
import functools

import jax
import jax.numpy as jnp
from jax import lax
from jax.experimental import pallas as pl
from jax.experimental.pallas import tpu as pltpu
from jax.experimental.pallas import tpu_sc as plsc

_NC = 2
_NS = 16
_NW = _NC * _NS
_WG = 16
_L = 16


def _transpose_block(src, dst, width, dim, col_iota):
    for d in range(dim):
        for l0 in range(0, width, _L):
            vals = src[d, pl.ds(l0, _L)]
            plsc.store_scatter(dst, [col_iota + (l0 * dim + d)], vals)


@functools.partial(jax.jit, static_argnames=("vocab", "dim"))
def _sc_transpose(w_t, tail_flat, vocab, dim):
    n_full = vocab // 128
    tail_n = (vocab - n_full * 128) * dim
    base_n = n_full // _NW
    extra = n_full - base_n * _NW
    mesh = plsc.VectorSubcoreMesh(core_axis_name="c", subcore_axis_name="s")

    def body(w_hbm, tail_hbm, out_hbm, buf, tbuf, sem):
        wid = lax.axis_index("s") * _NC + lax.axis_index("c")
        my_n = jnp.where(wid < extra, base_n + 1, base_n)
        start = wid * base_n + jnp.minimum(wid, extra)
        col_iota = lax.iota(jnp.int32, _L) * dim

        def step(i, carry):
            vt = start + i
            src_col = pl.multiple_of(vt * 128, 128)
            dst_off = pl.multiple_of(vt * (128 * dim), 128 * dim)
            pltpu.async_copy(
                w_hbm.at[:, pl.ds(src_col, 128)], buf, sem
            ).wait()
            _transpose_block(buf, tbuf, 128, dim, col_iota)
            pltpu.async_copy(
                tbuf, out_hbm.at[pl.ds(dst_off, 128 * dim)], sem
            ).wait()
            return carry

        lax.fori_loop(0, my_n, step, 0)

        if tail_n:
            @pl.when(wid == _NW - 1)
            def _tail():
                pltpu.async_copy(
                    tail_hbm, tbuf.at[pl.ds(0, tail_n)], sem
                ).wait()
                pltpu.async_copy(
                    tbuf.at[pl.ds(0, tail_n)],
                    out_hbm.at[pl.ds(n_full * 128 * dim, tail_n)],
                    sem,
                ).wait()

    fn = pl.kernel(
        body,
        out_type=jax.ShapeDtypeStruct((vocab * dim,), jnp.float32),
        mesh=mesh,
        scratch_types=[
            pltpu.VMEM((dim, 128), jnp.float32),
            pltpu.VMEM((128 * dim,), jnp.float32),
            pltpu.SemaphoreType.DMA,
        ],
        compiler_params=pltpu.CompilerParams(needs_layout_passes=False),
    )
    return fn(w_t, tail_flat)


@functools.partial(jax.jit, static_argnames=("rows_per_w", "hist", "dim"))
def _sc_embed(table, idx, rows_per_w, hist, dim):
    batch = rows_per_w * _NW
    n_groups = rows_per_w // _WG
    mesh = plsc.VectorSubcoreMesh(core_axis_name="c", subcore_axis_name="s")

    def body(w_hbm, idx_hbm, out_hbm, idx_v, wide, gsem, wsem):
        wid = lax.axis_index("s") * _NC + lax.axis_index("c")
        base = wid * rows_per_w
        pltpu.sync_copy(idx_hbm.at[pl.ds(base, rows_per_w)], idx_v)

        def group(g, carry):
            r0 = g * _WG
            for k in range(_WG):
                pltpu.async_copy(w_hbm.at[idx_v.at[r0 + k]], wide.at[k], gsem)
            for k in range(_WG):
                pltpu.make_async_copy(
                    w_hbm.at[idx_v.at[r0 + k]], wide.at[k], gsem
                ).wait()
            copy = pltpu.make_async_copy(
                wide, out_hbm.at[pl.ds(base + r0, _WG)], wsem
            )
            copy.start()
            copy.wait()
            return carry

        lax.fori_loop(0, n_groups, group, 0)

    fn = pl.kernel(
        body,
        out_type=jax.ShapeDtypeStruct((batch, hist, dim), jnp.float32),
        mesh=mesh,
        scratch_types=[
            pltpu.VMEM((rows_per_w, hist), jnp.int32),
            pltpu.VMEM((_WG, hist, dim), jnp.float32),
            pltpu.SemaphoreType.DMA,
            pltpu.SemaphoreType.DMA,
        ],
        compiler_params=pltpu.CompilerParams(use_tc_tiling_on_sc=False),
    )
    return fn(table, idx)


def kernel(input_variable, weight):
    vocab, dim = weight.shape
    batch, hist = input_variable.shape
    idx = input_variable.astype(jnp.int32)
    grain = _NW * _WG
    batch_pad = -(-batch // grain) * grain
    if batch_pad != batch:
        idx = jnp.pad(idx, ((0, batch_pad - batch), (0, 0)))
    n_full = vocab // 128
    if vocab % 128:
        tail_flat = weight[n_full * 128:].reshape(-1)
    else:
        tail_flat = weight[:1].reshape(-1)
    flat = _sc_transpose(jnp.transpose(weight), tail_flat, vocab, dim)
    table = flat.reshape(vocab, dim)
    out = _sc_embed(table, idx, batch_pad // _NW, hist, dim)
    if batch_pad != batch:
        out = out[:batch]
    return out


# speedup vs baseline: 1.1041x; 1.1041x over previous
import functools

import jax
import jax.numpy as jnp
from jax import lax
from jax.experimental import pallas as pl
from jax.experimental.pallas import tpu as pltpu
from jax.experimental.pallas import tpu_sc as plsc

_NC = 2
_NS = 16
_NW = _NC * _NS
_WG = 16
_L = 16


def _transpose_block(src, dst, width, dim, col_iota):
    for d in range(dim):
        for l0 in range(0, width, _L):
            vals = src[d, pl.ds(l0, _L)]
            plsc.store_scatter(dst, [col_iota + (l0 * dim + d)], vals)


@functools.partial(jax.jit, static_argnames=("vocab", "dim"))
def _sc_transpose(w_t, tail_flat, vocab, dim):
    n_full = vocab // 128
    tail_n = (vocab - n_full * 128) * dim
    n_pairs = n_full // 2
    odd_vt = n_full - 1 if n_full % 2 else -1
    base_p = n_pairs // _NW
    extra_p = n_pairs - base_p * _NW
    mesh = plsc.VectorSubcoreMesh(core_axis_name="c", subcore_axis_name="s")

    def body(w_hbm, tail_hbm, out_hbm, buf0, buf1, tbuf0, tbuf1,
             rsem0, rsem1, wsem0, wsem1):
        wid = lax.axis_index("s") * _NC + lax.axis_index("c")
        my_np = jnp.where(wid < extra_p, base_p + 1, base_p)
        start_p = wid * base_p + jnp.minimum(wid, extra_p)
        col_iota = lax.iota(jnp.int32, _L) * dim
        bufs = (buf0, buf1)
        tbufs = (tbuf0, tbuf1)
        rsems = (rsem0, rsem1)
        wsems = (wsem0, wsem1)

        def read(vt, b):
            src_col = pl.multiple_of(vt * 128, 128)
            pltpu.async_copy(w_hbm.at[:, pl.ds(src_col, 128)], bufs[b],
                             rsems[b])

        def wait_read(b):
            pltpu.make_async_copy(w_hbm.at[:, pl.ds(0, 128)], bufs[b],
                                  rsems[b]).wait()

        def write(vt, b):
            dst_off = pl.multiple_of(vt * (128 * dim), 128 * dim)
            pltpu.async_copy(tbufs[b],
                             out_hbm.at[pl.ds(dst_off, 128 * dim)],
                             wsems[b])

        def wait_write(b):
            pltpu.make_async_copy(tbufs[b],
                                  out_hbm.at[pl.ds(0, 128 * dim)],
                                  wsems[b]).wait()

        @pl.when(my_np > 0)
        def _prime():
            read(2 * start_p, 0)

        def step(p, carry):
            vt0 = 2 * (start_p + p)
            read(vt0 + 1, 1)
            wait_read(0)

            @pl.when(p > 0)
            def _():
                wait_write(0)

            _transpose_block(bufs[0], tbufs[0], 128, dim, col_iota)
            write(vt0, 0)

            @pl.when(p + 1 < my_np)
            def _():
                read(vt0 + 2, 0)

            wait_read(1)

            @pl.when(p > 0)
            def _():
                wait_write(1)

            _transpose_block(bufs[1], tbufs[1], 128, dim, col_iota)
            write(vt0 + 1, 1)
            return carry

        lax.fori_loop(0, my_np, step, 0)

        @pl.when(my_np > 0)
        def _drain():
            wait_write(0)
            wait_write(1)

        if odd_vt >= 0:
            @pl.when(wid == 0)
            def _odd():
                read(odd_vt, 0)
                wait_read(0)
                _transpose_block(bufs[0], tbufs[0], 128, dim, col_iota)
                write(odd_vt, 0)
                wait_write(0)

        if tail_n:
            @pl.when(wid == _NW - 1)
            def _tail():
                pltpu.async_copy(
                    tail_hbm, tbuf0.at[pl.ds(0, tail_n)], rsem0
                ).wait()
                pltpu.async_copy(
                    tbuf0.at[pl.ds(0, tail_n)],
                    out_hbm.at[pl.ds(n_full * 128 * dim, tail_n)],
                    rsem0,
                ).wait()

    fn = pl.kernel(
        body,
        out_type=jax.ShapeDtypeStruct((vocab * dim,), jnp.float32),
        mesh=mesh,
        scratch_types=[
            pltpu.VMEM((dim, 128), jnp.float32),
            pltpu.VMEM((dim, 128), jnp.float32),
            pltpu.VMEM((128 * dim,), jnp.float32),
            pltpu.VMEM((128 * dim,), jnp.float32),
            pltpu.SemaphoreType.DMA,
            pltpu.SemaphoreType.DMA,
            pltpu.SemaphoreType.DMA,
            pltpu.SemaphoreType.DMA,
        ],
        compiler_params=pltpu.CompilerParams(needs_layout_passes=False),
    )
    return fn(w_t, tail_flat)


@functools.partial(jax.jit, static_argnames=("rows_per_w", "hist", "dim", "hc_sz"))
def _sc_embed_tiled(table, idx5, rows_per_w, hist, dim, hc_sz):
    batch = rows_per_w * _NW
    nbt = rows_per_w // 128
    n_hc = hist // hc_sz
    kd = dim // 8
    n_it = nbt * n_hc
    tr_per_it = 128 * hc_sz // 128
    n_tr = n_it * tr_per_it
    g_rows = 128 * hc_sz
    mesh = plsc.VectorSubcoreMesh(core_axis_name="c", subcore_axis_name="s")

    def body(w_hbm, idx_hbm, out_hbm, idx_v, g0, g1, st, gsem0, gsem1, wsem):
        wid = lax.axis_index("s") * _NC + lax.axis_index("c")
        pltpu.sync_copy(idx_hbm.at[wid], idx_v)
        iota = lax.iota(jnp.int32, _L)
        iota_h = iota * hc_sz
        gs = (g0, g1)
        gsems = (gsem0, gsem1)

        def fire(it, b):
            t0 = it * tr_per_it
            for j in range(tr_per_it):
                pltpu.async_copy(
                    w_hbm.at[idx_v.at[t0 + j]],
                    gs[b].at[pl.ds(j * 128, 128)],
                    gsems[b],
                )

        def drain(b):
            for j in range(tr_per_it):
                pltpu.make_async_copy(
                    w_hbm.at[idx_v.at[0]],
                    gs[b].at[pl.ds(0, 128)],
                    gsems[b],
                ).wait()

        def permute(b):
            def ph(hh, carry):
                rows0 = iota_h + hh
                for d in range(dim):
                    cols = jnp.full((_L,), d, jnp.int32)
                    k, sub = d // 8, d % 8
                    for oct in range(8):
                        rows = rows0 + (oct * _L * hc_sz)
                        vals = plsc.load_gather(gs[b], [rows, cols])
                        st[hh, k, sub, pl.ds(oct * _L, _L)] = vals
                return carry
            lax.fori_loop(0, hc_sz, ph, 0)

        def wait_write():
            pltpu.make_async_copy(
                st, out_hbm.at[pl.ds(0, hc_sz), :, 0], wsem
            ).wait()

        def write(it):
            bt = it // n_hc
            hc = it - bt * n_hc
            btg = wid * nbt + bt
            pltpu.async_copy(
                st, out_hbm.at[pl.ds(hc * hc_sz, hc_sz), :, btg], wsem
            )

        fire(0, 0)

        def step(q, carry):
            it_a = 2 * q
            fire(it_a + 1, 1)
            drain(0)

            @pl.when(q > 0)
            def _():
                wait_write()

            permute(0)
            write(it_a)

            @pl.when(q + 1 < n_it // 2)
            def _():
                fire(it_a + 2, 0)

            drain(1)
            wait_write()
            permute(1)
            write(it_a + 1)
            return carry

        lax.fori_loop(0, n_it // 2, step, 0)
        wait_write()

    fn = pl.kernel(
        body,
        out_type=jax.ShapeDtypeStruct(
            (hist, kd, batch // 128, 8, 128), jnp.float32
        ),
        mesh=mesh,
        scratch_types=[
            pltpu.VMEM((n_tr, 128), jnp.int32),
            pltpu.VMEM((g_rows, dim), jnp.float32),
            pltpu.VMEM((g_rows, dim), jnp.float32),
            pltpu.VMEM((hc_sz, kd, 8, 128), jnp.float32),
            pltpu.SemaphoreType.DMA,
            pltpu.SemaphoreType.DMA,
            pltpu.SemaphoreType.DMA,
        ],
        compiler_params=pltpu.CompilerParams(
            use_tc_tiling_on_sc=False, needs_layout_passes=False
        ),
    )
    return fn(table, idx5)


def kernel(input_variable, weight):
    vocab, dim = weight.shape
    batch, hist = input_variable.shape
    idx = input_variable.astype(jnp.int32)
    grain = _NW * 256
    batch_pad = -(-batch // grain) * grain
    if batch_pad != batch:
        idx = jnp.pad(idx, ((0, batch_pad - batch), (0, 0)))
    n_full = vocab // 128
    if vocab % 128:
        tail_flat = weight[n_full * 128:].reshape(-1)
    else:
        tail_flat = weight[:1].reshape(-1)
    flat = _sc_transpose(jnp.transpose(weight), tail_flat, vocab, dim)
    table = flat.reshape(vocab, dim)

    hc_sz = 5 if hist % 5 == 0 else 1
    rows_per_w = batch_pad // _NW
    nbt = rows_per_w // 128
    idx5 = (
        idx.reshape(_NW, nbt, 128, hist // hc_sz, hc_sz)
        .transpose(0, 1, 3, 2, 4)
        .reshape(_NW, -1, 128)
    )
    out5 = _sc_embed_tiled(table, idx5, rows_per_w, hist, dim, hc_sz)
    out = out5.transpose(2, 4, 0, 1, 3).reshape(batch_pad, hist, dim)
    if batch_pad != batch:
        out = out[:batch]
    return out


# speedup vs baseline: 1.7165x; 1.5546x over previous
import functools

import jax
import jax.numpy as jnp
from jax import lax
from jax.experimental import pallas as pl
from jax.experimental.pallas import tpu as pltpu
from jax.experimental.pallas import tpu_sc as plsc

_NC = 2
_NS = 16
_NW = _NC * _NS
_WG = 16
_L = 16


def _transpose_block(src, dst, width, dim, col_iota):
    @plsc.parallel_loop(0, dim, step=1, unroll=4)
    def _pd(d):
        for l0 in range(0, width, _L):
            vals = src[d, pl.ds(l0, _L)]
            plsc.store_scatter(dst, [col_iota + (l0 * dim) + d], vals)


@functools.partial(jax.jit, static_argnames=("vocab", "dim"))
def _sc_transpose(w_t, tail_flat, vocab, dim):
    n_full = vocab // 128
    tail_n = (vocab - n_full * 128) * dim
    n_pairs = n_full // 2
    odd_vt = n_full - 1 if n_full % 2 else -1
    base_p = n_pairs // _NW
    extra_p = n_pairs - base_p * _NW
    mesh = plsc.VectorSubcoreMesh(core_axis_name="c", subcore_axis_name="s")

    def body(w_hbm, tail_hbm, out_hbm, buf0, buf1, tbuf0, tbuf1,
             rsem0, rsem1, wsem0, wsem1):
        wid = lax.axis_index("s") * _NC + lax.axis_index("c")
        my_np = jnp.where(wid < extra_p, base_p + 1, base_p)
        start_p = wid * base_p + jnp.minimum(wid, extra_p)
        col_iota = lax.iota(jnp.int32, _L) * dim
        bufs = (buf0, buf1)
        tbufs = (tbuf0, tbuf1)
        rsems = (rsem0, rsem1)
        wsems = (wsem0, wsem1)

        def read(vt, b):
            src_col = pl.multiple_of(vt * 128, 128)
            pltpu.async_copy(w_hbm.at[:, pl.ds(src_col, 128)], bufs[b],
                             rsems[b])

        def wait_read(b):
            pltpu.make_async_copy(w_hbm.at[:, pl.ds(0, 128)], bufs[b],
                                  rsems[b]).wait()

        def write(vt, b):
            dst_off = pl.multiple_of(vt * (128 * dim), 128 * dim)
            pltpu.async_copy(tbufs[b],
                             out_hbm.at[pl.ds(dst_off, 128 * dim)],
                             wsems[b])

        def wait_write(b):
            pltpu.make_async_copy(tbufs[b],
                                  out_hbm.at[pl.ds(0, 128 * dim)],
                                  wsems[b]).wait()

        @pl.when(my_np > 0)
        def _prime():
            read(2 * start_p, 0)

        def step(p, carry):
            vt0 = 2 * (start_p + p)
            read(vt0 + 1, 1)
            wait_read(0)

            @pl.when(p > 0)
            def _():
                wait_write(0)

            _transpose_block(bufs[0], tbufs[0], 128, dim, col_iota)
            write(vt0, 0)

            @pl.when(p + 1 < my_np)
            def _():
                read(vt0 + 2, 0)

            wait_read(1)

            @pl.when(p > 0)
            def _():
                wait_write(1)

            _transpose_block(bufs[1], tbufs[1], 128, dim, col_iota)
            write(vt0 + 1, 1)
            return carry

        lax.fori_loop(0, my_np, step, 0)

        @pl.when(my_np > 0)
        def _drain():
            wait_write(0)
            wait_write(1)

        if odd_vt >= 0:
            @pl.when(wid == 0)
            def _odd():
                read(odd_vt, 0)
                wait_read(0)
                _transpose_block(bufs[0], tbufs[0], 128, dim, col_iota)
                write(odd_vt, 0)
                wait_write(0)

        if tail_n:
            @pl.when(wid == _NW - 1)
            def _tail():
                pltpu.async_copy(
                    tail_hbm, tbuf0.at[pl.ds(0, tail_n)], rsem0
                ).wait()
                pltpu.async_copy(
                    tbuf0.at[pl.ds(0, tail_n)],
                    out_hbm.at[pl.ds(n_full * 128 * dim, tail_n)],
                    rsem0,
                ).wait()

    fn = pl.kernel(
        body,
        out_type=jax.ShapeDtypeStruct((vocab * dim,), jnp.float32),
        mesh=mesh,
        scratch_types=[
            pltpu.VMEM((dim, 128), jnp.float32),
            pltpu.VMEM((dim, 128), jnp.float32),
            pltpu.VMEM((128 * dim,), jnp.float32),
            pltpu.VMEM((128 * dim,), jnp.float32),
            pltpu.SemaphoreType.DMA,
            pltpu.SemaphoreType.DMA,
            pltpu.SemaphoreType.DMA,
            pltpu.SemaphoreType.DMA,
        ],
        compiler_params=pltpu.CompilerParams(needs_layout_passes=False),
    )
    return fn(w_t, tail_flat)


@functools.partial(jax.jit, static_argnames=("rows_per_w", "hist", "dim", "hc_sz"))
def _sc_embed_tiled(table, idx5, rows_per_w, hist, dim, hc_sz):
    batch = rows_per_w * _NW
    nbt = rows_per_w // 128
    n_hc = hist // hc_sz
    kd = dim // 8
    n_it = nbt * n_hc
    tr_per_it = 128 * hc_sz // 128
    n_tr = n_it * tr_per_it
    g_rows = 128 * hc_sz
    mesh = plsc.VectorSubcoreMesh(core_axis_name="c", subcore_axis_name="s")

    def body(w_hbm, idx_hbm, out_hbm, idx_v, g0, g1, st, gsem0, gsem1, wsem):
        wid = lax.axis_index("s") * _NC + lax.axis_index("c")
        pltpu.sync_copy(idx_hbm.at[wid], idx_v)
        iota = lax.iota(jnp.int32, _L)
        iota_h = iota * hc_sz
        gs = (g0, g1)
        gsems = (gsem0, gsem1)

        def fire(it, b):
            t0 = it * tr_per_it
            for j in range(tr_per_it):
                pltpu.async_copy(
                    w_hbm.at[idx_v.at[t0 + j]],
                    gs[b].at[pl.ds(j * 128, 128)],
                    gsems[b],
                )

        def drain(b):
            for j in range(tr_per_it):
                pltpu.make_async_copy(
                    w_hbm.at[idx_v.at[0]],
                    gs[b].at[pl.ds(0, 128)],
                    gsems[b],
                ).wait()

        def permute(b):
            @plsc.parallel_loop(0, hc_sz * dim, step=1, unroll=4)
            def _pi(i):
                hh = i // dim
                d = i - hh * dim
                k = d // 8
                sub = d - k * 8
                cols = jnp.full((_L,), 0, jnp.int32) + d
                rows0 = iota_h + hh
                for oct in range(8):
                    rows = rows0 + (oct * _L * hc_sz)
                    vals = plsc.load_gather(gs[b], [rows, cols])
                    st[hh, k, sub, pl.ds(oct * _L, _L)] = vals

        def wait_write():
            pltpu.make_async_copy(
                st, out_hbm.at[pl.ds(0, hc_sz), :, 0], wsem
            ).wait()

        def write(it):
            bt = it // n_hc
            hc = it - bt * n_hc
            btg = wid * nbt + bt
            pltpu.async_copy(
                st, out_hbm.at[pl.ds(hc * hc_sz, hc_sz), :, btg], wsem
            )

        fire(0, 0)

        def step(q, carry):
            it_a = 2 * q
            fire(it_a + 1, 1)
            drain(0)

            @pl.when(q > 0)
            def _():
                wait_write()

            permute(0)
            write(it_a)

            @pl.when(q + 1 < n_it // 2)
            def _():
                fire(it_a + 2, 0)

            drain(1)
            wait_write()
            permute(1)
            write(it_a + 1)
            return carry

        lax.fori_loop(0, n_it // 2, step, 0)
        wait_write()

    fn = pl.kernel(
        body,
        out_type=jax.ShapeDtypeStruct(
            (hist, kd, batch // 128, 8, 128), jnp.float32
        ),
        mesh=mesh,
        scratch_types=[
            pltpu.VMEM((n_tr, 128), jnp.int32),
            pltpu.VMEM((g_rows, dim), jnp.float32),
            pltpu.VMEM((g_rows, dim), jnp.float32),
            pltpu.VMEM((hc_sz, kd, 8, 128), jnp.float32),
            pltpu.SemaphoreType.DMA,
            pltpu.SemaphoreType.DMA,
            pltpu.SemaphoreType.DMA,
        ],
        compiler_params=pltpu.CompilerParams(
            use_tc_tiling_on_sc=False, needs_layout_passes=False
        ),
    )
    return fn(table, idx5)


def kernel(input_variable, weight):
    vocab, dim = weight.shape
    batch, hist = input_variable.shape
    idx = input_variable.astype(jnp.int32)
    grain = _NW * 256
    batch_pad = -(-batch // grain) * grain
    if batch_pad != batch:
        idx = jnp.pad(idx, ((0, batch_pad - batch), (0, 0)))
    n_full = vocab // 128
    if vocab % 128:
        tail_flat = weight[n_full * 128:].reshape(-1)
    else:
        tail_flat = weight[:1].reshape(-1)
    flat = _sc_transpose(jnp.transpose(weight), tail_flat, vocab, dim)
    table = flat.reshape(vocab, dim)

    hc_sz = 5 if hist % 5 == 0 else 1
    rows_per_w = batch_pad // _NW
    nbt = rows_per_w // 128
    idx5 = (
        idx.reshape(_NW, nbt, 128, hist // hc_sz, hc_sz)
        .transpose(0, 1, 3, 2, 4)
        .reshape(_NW, -1, 128)
    )
    out5 = _sc_embed_tiled(table, idx5, rows_per_w, hist, dim, hc_sz)
    out = out5.transpose(2, 4, 0, 1, 3).reshape(batch_pad, hist, dim)
    if batch_pad != batch:
        out = out[:batch]
    return out


# speedup vs baseline: 1.7476x; 1.0181x over previous
import functools

import jax
import jax.numpy as jnp
from jax import lax
from jax.experimental import pallas as pl
from jax.experimental.pallas import tpu as pltpu
from jax.experimental.pallas import tpu_sc as plsc

_NC = 2
_NS = 16
_NW = _NC * _NS
_WG = 16
_L = 16


def _transpose_block(src, dst, width, dim, col_iota):
    @plsc.parallel_loop(0, dim, step=1, unroll=8)
    def _pd(d):
        for l0 in range(0, width, _L):
            vals = src[d, pl.ds(l0, _L)]
            plsc.store_scatter(dst, [col_iota + (l0 * dim) + d], vals)


@functools.partial(jax.jit, static_argnames=("vocab", "dim"))
def _sc_transpose(w_t, tail_flat, vocab, dim):
    n_full = vocab // 128
    tail_n = (vocab - n_full * 128) * dim
    n_pairs = n_full // 2
    odd_vt = n_full - 1 if n_full % 2 else -1
    base_p = n_pairs // _NW
    extra_p = n_pairs - base_p * _NW
    mesh = plsc.VectorSubcoreMesh(core_axis_name="c", subcore_axis_name="s")

    def body(w_hbm, tail_hbm, out_hbm, buf0, buf1, tbuf0, tbuf1,
             rsem0, rsem1, wsem0, wsem1):
        wid = lax.axis_index("s") * _NC + lax.axis_index("c")
        my_np = jnp.where(wid < extra_p, base_p + 1, base_p)
        start_p = wid * base_p + jnp.minimum(wid, extra_p)
        col_iota = lax.iota(jnp.int32, _L) * dim
        bufs = (buf0, buf1)
        tbufs = (tbuf0, tbuf1)
        rsems = (rsem0, rsem1)
        wsems = (wsem0, wsem1)

        def read(vt, b):
            src_col = pl.multiple_of(vt * 128, 128)
            pltpu.async_copy(w_hbm.at[:, pl.ds(src_col, 128)], bufs[b],
                             rsems[b])

        def wait_read(b):
            pltpu.make_async_copy(w_hbm.at[:, pl.ds(0, 128)], bufs[b],
                                  rsems[b]).wait()

        def write(vt, b):
            dst_off = pl.multiple_of(vt * (128 * dim), 128 * dim)
            pltpu.async_copy(tbufs[b],
                             out_hbm.at[pl.ds(dst_off, 128 * dim)],
                             wsems[b])

        def wait_write(b):
            pltpu.make_async_copy(tbufs[b],
                                  out_hbm.at[pl.ds(0, 128 * dim)],
                                  wsems[b]).wait()

        @pl.when(my_np > 0)
        def _prime():
            read(2 * start_p, 0)

        def step(p, carry):
            vt0 = 2 * (start_p + p)
            read(vt0 + 1, 1)
            wait_read(0)

            @pl.when(p > 0)
            def _():
                wait_write(0)

            _transpose_block(bufs[0], tbufs[0], 128, dim, col_iota)
            write(vt0, 0)

            @pl.when(p + 1 < my_np)
            def _():
                read(vt0 + 2, 0)

            wait_read(1)

            @pl.when(p > 0)
            def _():
                wait_write(1)

            _transpose_block(bufs[1], tbufs[1], 128, dim, col_iota)
            write(vt0 + 1, 1)
            return carry

        lax.fori_loop(0, my_np, step, 0)

        @pl.when(my_np > 0)
        def _drain():
            wait_write(0)
            wait_write(1)

        if odd_vt >= 0:
            @pl.when(wid == 0)
            def _odd():
                read(odd_vt, 0)
                wait_read(0)
                _transpose_block(bufs[0], tbufs[0], 128, dim, col_iota)
                write(odd_vt, 0)
                wait_write(0)

        if tail_n:
            @pl.when(wid == _NW - 1)
            def _tail():
                pltpu.async_copy(
                    tail_hbm, tbuf0.at[pl.ds(0, tail_n)], rsem0
                ).wait()
                pltpu.async_copy(
                    tbuf0.at[pl.ds(0, tail_n)],
                    out_hbm.at[pl.ds(n_full * 128 * dim, tail_n)],
                    rsem0,
                ).wait()

    fn = pl.kernel(
        body,
        out_type=jax.ShapeDtypeStruct((vocab * dim,), jnp.float32),
        mesh=mesh,
        scratch_types=[
            pltpu.VMEM((dim, 128), jnp.float32),
            pltpu.VMEM((dim, 128), jnp.float32),
            pltpu.VMEM((128 * dim,), jnp.float32),
            pltpu.VMEM((128 * dim,), jnp.float32),
            pltpu.SemaphoreType.DMA,
            pltpu.SemaphoreType.DMA,
            pltpu.SemaphoreType.DMA,
            pltpu.SemaphoreType.DMA,
        ],
        compiler_params=pltpu.CompilerParams(needs_layout_passes=False),
    )
    return fn(w_t, tail_flat)


@functools.partial(jax.jit, static_argnames=("rows_per_w", "hist", "dim", "hc_sz"))
def _sc_embed_tiled(table, idx5, rows_per_w, hist, dim, hc_sz):
    batch = rows_per_w * _NW
    nbt = rows_per_w // 128
    n_hc = hist // hc_sz
    kd = dim // 8
    n_it = nbt * n_hc
    tr_per_it = 128 * hc_sz // 128
    n_tr = n_it * tr_per_it
    g_rows = 128 * hc_sz
    mesh = plsc.VectorSubcoreMesh(core_axis_name="c", subcore_axis_name="s")

    def body(w_hbm, idx_hbm, out_hbm, idx_v, g0, g1, st, gsem0, gsem1, wsem):
        wid = lax.axis_index("s") * _NC + lax.axis_index("c")
        pltpu.sync_copy(idx_hbm.at[wid], idx_v)
        iota = lax.iota(jnp.int32, _L)
        iota_h = iota * hc_sz
        gs = (g0, g1)
        gsems = (gsem0, gsem1)

        def fire(it, b):
            t0 = it * tr_per_it
            for j in range(tr_per_it):
                pltpu.async_copy(
                    w_hbm.at[idx_v.at[t0 + j]],
                    gs[b].at[pl.ds(j * 128, 128)],
                    gsems[b],
                )

        def drain(b):
            for j in range(tr_per_it):
                pltpu.make_async_copy(
                    w_hbm.at[idx_v.at[0]],
                    gs[b].at[pl.ds(0, 128)],
                    gsems[b],
                ).wait()

        def permute(b):
            @plsc.parallel_loop(0, hc_sz * dim, step=1, unroll=8)
            def _pi(i):
                hh = i // dim
                d = i - hh * dim
                k = d // 8
                sub = d - k * 8
                cols = jnp.full((_L,), 0, jnp.int32) + d
                rows0 = iota_h + hh
                for oct in range(8):
                    rows = rows0 + (oct * _L * hc_sz)
                    vals = plsc.load_gather(gs[b], [rows, cols])
                    st[hh, k, sub, pl.ds(oct * _L, _L)] = vals

        def wait_write():
            pltpu.make_async_copy(
                st, out_hbm.at[pl.ds(0, hc_sz), :, 0], wsem
            ).wait()

        def write(it):
            bt = it // n_hc
            hc = it - bt * n_hc
            btg = wid * nbt + bt
            pltpu.async_copy(
                st, out_hbm.at[pl.ds(hc * hc_sz, hc_sz), :, btg], wsem
            )

        fire(0, 0)

        def step(q, carry):
            it_a = 2 * q
            fire(it_a + 1, 1)
            drain(0)

            @pl.when(q > 0)
            def _():
                wait_write()

            permute(0)
            write(it_a)

            @pl.when(q + 1 < n_it // 2)
            def _():
                fire(it_a + 2, 0)

            drain(1)
            wait_write()
            permute(1)
            write(it_a + 1)
            return carry

        lax.fori_loop(0, n_it // 2, step, 0)
        wait_write()

    fn = pl.kernel(
        body,
        out_type=jax.ShapeDtypeStruct(
            (hist, kd, batch // 128, 8, 128), jnp.float32
        ),
        mesh=mesh,
        scratch_types=[
            pltpu.VMEM((n_tr, 128), jnp.int32),
            pltpu.VMEM((g_rows, dim), jnp.float32),
            pltpu.VMEM((g_rows, dim), jnp.float32),
            pltpu.VMEM((hc_sz, kd, 8, 128), jnp.float32),
            pltpu.SemaphoreType.DMA,
            pltpu.SemaphoreType.DMA,
            pltpu.SemaphoreType.DMA,
        ],
        compiler_params=pltpu.CompilerParams(
            use_tc_tiling_on_sc=False, needs_layout_passes=False
        ),
    )
    return fn(table, idx5)


def kernel(input_variable, weight):
    vocab, dim = weight.shape
    batch, hist = input_variable.shape
    idx = input_variable.astype(jnp.int32)
    grain = _NW * 256
    batch_pad = -(-batch // grain) * grain
    if batch_pad != batch:
        idx = jnp.pad(idx, ((0, batch_pad - batch), (0, 0)))
    n_full = vocab // 128
    if vocab % 128:
        tail_flat = weight[n_full * 128:].reshape(-1)
    else:
        tail_flat = weight[:1].reshape(-1)
    flat = _sc_transpose(jnp.transpose(weight), tail_flat, vocab, dim)
    table = flat.reshape(vocab, dim)

    hc_sz = 5 if hist % 5 == 0 else 1
    rows_per_w = batch_pad // _NW
    nbt = rows_per_w // 128
    idx5 = (
        idx.reshape(_NW, nbt, 128, hist // hc_sz, hc_sz)
        .transpose(0, 1, 3, 2, 4)
        .reshape(_NW, -1, 128)
    )
    out5 = _sc_embed_tiled(table, idx5, rows_per_w, hist, dim, hc_sz)
    out = out5.transpose(2, 4, 0, 1, 3).reshape(batch_pad, hist, dim)
    if batch_pad != batch:
        out = out[:batch]
    return out
